# baseline (speedup 1.0000x reference)
import functools

import jax
import jax.numpy as jnp
from jax import lax
from jax.experimental import pallas as pl
from jax.experimental.pallas import tpu as pltpu
from jax.experimental.pallas import tpu_sc as plsc

NUM_FIELDS = 26
VOCAB = 100000
BATCH = 16384
DIM = 64
LANES = 16

NUM_CORES = 2
NUM_SUBCORES = 16
NUM_WORKERS = NUM_CORES * NUM_SUBCORES
BPW = BATCH // NUM_WORKERS
CHUNK = 128
NCH = BPW // CHUNK
ROW_UNROLL = 8


def _accumulate(acc, buf):
    def body(r, carry):
        for dr in range(ROW_UNROLL):
            row = r * ROW_UNROLL + dr
            for c in range(DIM // LANES):
                sl = pl.ds(c * LANES, LANES)
                plsc.addupdate(acc.at[row, sl], buf[row, sl])
        return carry

    lax.fori_loop(0, BPW // ROW_UNROLL, body, 0)


def _body(*refs):
    ins = refs[: 2 * NUM_FIELDS]
    out = refs[2 * NUM_FIELDS]
    idx, acc, buf_a, buf_b, sem_idx, sem_a, sem_b, sem_acc = refs[2 * NUM_FIELDS + 1 :]
    cats = ins[0::2]
    tables = ins[1::2]

    wid = lax.axis_index("c") * NUM_SUBCORES + lax.axis_index("s")
    base = wid * BPW

    pend = []
    for i in range(NUM_FIELDS):
        for j in range(NCH):
            d = pltpu.async_copy(
                cats[i].at[pl.ds(base + j * CHUNK, CHUNK)],
                idx.at[i * NCH + j],
                sem_idx,
            )
            pend.append(d)
            if len(pend) == 8:
                for d2 in pend:
                    d2.wait()
                pend = []
    for d2 in pend:
        d2.wait()

    def start_field(i, dst, sem):
        descs = []
        for j in range(NCH):
            idx_view = idx.at[i * NCH + j]
            descs.append(
                pltpu.async_copy(
                    tables[i].at[idx_view],
                    dst.at[pl.ds(j * CHUNK, CHUNK)],
                    sem,
                )
            )
        return descs

    bufs = (buf_a, buf_b)
    sems = (sem_a, sem_b)

    d_acc = start_field(0, acc, sem_acc)
    inflight = [start_field(1, buf_a, sem_a), None]
    for d in d_acc:
        d.wait()
    inflight[1] = start_field(2, buf_b, sem_b)

    for i in range(1, NUM_FIELDS):
        b = (i - 1) % 2
        for d in inflight[b]:
            d.wait()
        _accumulate(acc, bufs[b])
        nxt = i + 2
        if nxt < NUM_FIELDS:
            inflight[b] = start_field(nxt, bufs[b], sems[b])

    pltpu.sync_copy(acc, out.at[pl.ds(base, BPW)])


@functools.partial(
    pl.kernel,
    mesh=plsc.VectorSubcoreMesh(core_axis_name="c", subcore_axis_name="s"),
    compiler_params=pltpu.CompilerParams(use_tc_tiling_on_sc=False),
    out_type=jax.ShapeDtypeStruct((BATCH, DIM), jnp.float32),
    scratch_types=[
        pltpu.VMEM((NUM_FIELDS * NCH, CHUNK), jnp.int32),
        pltpu.VMEM((BPW, DIM), jnp.float32),
        pltpu.VMEM((BPW, DIM), jnp.float32),
        pltpu.VMEM((BPW, DIM), jnp.float32),
        pltpu.SemaphoreType.DMA,
        pltpu.SemaphoreType.DMA,
        pltpu.SemaphoreType.DMA,
        pltpu.SemaphoreType.DMA,
    ],
)
def _embed_sum(*refs):
    _body(*refs)


def kernel(cat_0, W_0, cat_1, W_1, cat_2, W_2, cat_3, W_3, cat_4, W_4, cat_5, W_5, cat_6, W_6, cat_7, W_7, cat_8, W_8, cat_9, W_9, cat_10, W_10, cat_11, W_11, cat_12, W_12, cat_13, W_13, cat_14, W_14, cat_15, W_15, cat_16, W_16, cat_17, W_17, cat_18, W_18, cat_19, W_19, cat_20, W_20, cat_21, W_21, cat_22, W_22, cat_23, W_23, cat_24, W_24, cat_25, W_25):
    args = locals()
    flat = []
    for i in range(NUM_FIELDS):
        flat.append(args[f"cat_{i}"])
        flat.append(args[f"W_{i}"])
    return _embed_sum(*flat)
